# baseline (speedup 1.0000x reference)
import functools
import numpy as np
import jax
import jax.numpy as jnp
from jax import lax
from jax.experimental import pallas as pl
from jax.experimental.pallas import tpu as pltpu
from jax.experimental.pallas import tpu_sc as plsc

_EPS = float(np.finfo(np.float32).eps)


def _knn_select(x2_ref, x1t_ref, idx_ref, w_ref, *, n_per_b):
    b = pl.program_id(0)
    x2 = x2_ref[0]
    x1t = x1t_ref[0]
    npts = x2.shape[0]
    n = x1t.shape[1]

    acc = None
    for d in range(3):
        diff = x2[:, d:d + 1] - x1t[d:d + 1, :]
        sq = diff * diff
        acc = sq if acc is None else acc + sq

    inf = jnp.float32(np.inf)
    iota = lax.broadcasted_iota(jnp.int32, (npts, n), 1)
    m1 = jnp.min(acc, axis=1, keepdims=True)
    d2 = jnp.where(acc == m1, inf, acc)
    m2 = jnp.min(d2, axis=1, keepdims=True)
    d3 = jnp.where(d2 == m2, inf, d2)
    m3 = jnp.min(d3, axis=1, keepdims=True)
    base = b * n_per_b
    a1 = jnp.min(jnp.where(acc == m1, iota, n), axis=1, keepdims=True) + base
    a2 = jnp.min(jnp.where(acc == m2, iota, n), axis=1, keepdims=True) + base
    a3 = jnp.min(jnp.where(acc == m3, iota, n), axis=1, keepdims=True) + base
    r1 = 1.0 / (m1 + _EPS)
    r2 = 1.0 / (m2 + _EPS)
    r3 = 1.0 / (m3 + _EPS)
    inv_tot = 1.0 / (r1 + r2 + r3)
    idx_ref[0] = jnp.concatenate([a1, a2, a3], axis=1)
    w_ref[0] = jnp.concatenate(
        [r1 * inv_tot, r2 * inv_tot, r3 * inv_tot], axis=1)


def _make_sc_interp(total_q, d1, q_chunk=16):
    info = plsc.get_sparse_core_info()
    nc, ns, lanes = info.num_cores, info.num_subcores, info.num_lanes
    nw = nc * ns
    nq_w = total_q // nw
    n_chunks = nq_w // q_chunk
    rows = 3 * q_chunk
    mesh = plsc.VectorSubcoreMesh(core_axis_name="c", subcore_axis_name="s")

    @functools.partial(
        pl.kernel, mesh=mesh,
        compiler_params=pltpu.CompilerParams(use_tc_tiling_on_sc=False),
        out_type=jax.ShapeDtypeStruct((total_q, d1), jnp.float32),
        scratch_types=[
            pltpu.VMEM((3 * nq_w,), jnp.int32),
            pltpu.VMEM((3 * nq_w, lanes), jnp.float32),
            pltpu.VMEM((rows, d1), jnp.float32),
            pltpu.VMEM((q_chunk, d1), jnp.float32),
            pltpu.SemaphoreType.DMA,
        ],
    )
    def sc_interp(table_hbm, idx_hbm, w16_hbm, out_hbm,
                  idx_v, w_v, rows_v, out_v, sem):
        wid = lax.axis_index("s") * nc + lax.axis_index("c")
        pltpu.sync_copy(idx_hbm.at[wid], idx_v)
        pltpu.sync_copy(w16_hbm.at[wid], w_v)

        def chunk_body(ci, _):
            cbase = ci * rows
            pltpu.async_copy(
                table_hbm.at[idx_v.at[pl.ds(cbase, rows)]], rows_v, sem
            ).wait()

            def q_body(qi, _):
                w0 = w_v[cbase + qi * 3, :]
                w1 = w_v[cbase + qi * 3 + 1, :]
                w2 = w_v[cbase + qi * 3 + 2, :]
                for j in range(d1 // lanes):
                    sl = pl.ds(j * lanes, lanes)
                    v = (w0 * rows_v[qi * 3, sl]
                         + w1 * rows_v[qi * 3 + 1, sl]
                         + w2 * rows_v[qi * 3 + 2, sl])
                    out_v[qi, sl] = v
                return 0

            lax.fori_loop(0, q_chunk, q_body, 0)
            pltpu.sync_copy(
                out_v,
                out_hbm.at[pl.ds(wid * nq_w + ci * q_chunk, q_chunk)])
            return 0

        lax.fori_loop(0, n_chunks, chunk_body, 0)

    return sc_interp


def _mlp(interp_ref, f2_ref, w0at_ref, w0bt_ref, w1t_ref,
         g0_ref, be0_ref, g1_ref, be1_ref, out_ref,
         y0_scr, y1_scr, s0_scr, ss0_scr, sc1_scr, sh1_scr,
         *, nb, count):
    p = pl.program_id(0)
    npts = f2_ref.shape[1]

    @pl.when(p < nb)
    def _phase_a():
        b = p
        y0 = jnp.dot(interp_ref[...], w0at_ref[...],
                     preferred_element_type=jnp.float32)
        y0 = y0 + jnp.dot(f2_ref[0], w0bt_ref[...],
                          preferred_element_type=jnp.float32)
        y0_scr[pl.ds(b * npts, npts)] = y0

        s = jnp.sum(y0, axis=0, keepdims=True)
        ss = jnp.sum(y0 * y0, axis=0, keepdims=True)

        @pl.when(b == 0)
        def _():
            s0_scr[...] = s
            ss0_scr[...] = ss

        @pl.when(b != 0)
        def _():
            s0_scr[...] += s
            ss0_scr[...] += ss

    @pl.when(p == nb)
    def _phase_b():
        mean = s0_scr[...] * (1.0 / count)
        var = ss0_scr[...] * (1.0 / count) - mean * mean
        inv = lax.rsqrt(var + 1e-5)
        scale = g0_ref[...] * inv
        shift = be0_ref[...] - mean * scale
        x = jnp.maximum(y0_scr[...] * scale + shift, 0.0)
        y1 = jnp.dot(x, w1t_ref[...], preferred_element_type=jnp.float32)
        y1_scr[...] = y1

        s1 = jnp.sum(y1, axis=0, keepdims=True)
        ss1 = jnp.sum(y1 * y1, axis=0, keepdims=True)
        mean1 = s1 * (1.0 / count)
        var1 = ss1 * (1.0 / count) - mean1 * mean1
        inv1 = lax.rsqrt(var1 + 1e-5)
        sc1_scr[...] = g1_ref[...] * inv1
        sh1_scr[...] = be1_ref[...] - mean1 * (g1_ref[...] * inv1)

    @pl.when(p > nb)
    def _phase_c():
        b = p - nb - 1
        out_ref[0] = jnp.maximum(
            y1_scr[pl.ds(b * npts, npts)] * sc1_scr[...] + sh1_scr[...], 0.0)


def kernel(xyz1, xyz2, features1, features2, W0, b0, g0, beta0,
           W1, b1, g1, beta1):
    B, N, _ = xyz1.shape
    Np = xyz2.shape[1]
    D1 = features1.shape[2]
    D2 = features2.shape[2]
    C0 = W0.shape[0]
    C1 = W1.shape[0]
    count = float(B * Np)
    total_q = B * Np

    x1t = xyz1.transpose(0, 2, 1)
    w0at = W0[:, :D1].T
    w0bt = W0[:, D1:].T
    w1t = W1.T
    g0r = g0.reshape(1, C0)
    beta0r = beta0.reshape(1, C0)
    g1r = g1.reshape(1, C1)
    beta1r = beta1.reshape(1, C1)

    idx, wn = pl.pallas_call(
        functools.partial(_knn_select, n_per_b=N),
        grid=(B,),
        in_specs=[
            pl.BlockSpec((1, Np, 3), lambda b: (b, 0, 0)),
            pl.BlockSpec((1, 3, N), lambda b: (b, 0, 0)),
        ],
        out_specs=[
            pl.BlockSpec((1, Np, 3), lambda b: (b, 0, 0)),
            pl.BlockSpec((1, Np, 3), lambda b: (b, 0, 0)),
        ],
        out_shape=[
            jax.ShapeDtypeStruct((B, Np, 3), jnp.int32),
            jax.ShapeDtypeStruct((B, Np, 3), jnp.float32),
        ],
        compiler_params=pltpu.CompilerParams(
            dimension_semantics=("arbitrary",)),
    )(xyz2, x1t)

    info = plsc.get_sparse_core_info()
    nw = info.num_cores * info.num_subcores
    lanes = info.num_lanes
    nq_w = total_q // nw
    idx_sc = idx.reshape(nw, nq_w * 3)
    w16 = jnp.broadcast_to(
        wn.reshape(total_q * 3, 1), (total_q * 3, lanes)
    ).reshape(nw, nq_w * 3, lanes)
    f1_flat = features1.reshape(B * N, D1)

    interp = _make_sc_interp(total_q, D1)(f1_flat, idx_sc, w16)

    out = pl.pallas_call(
        functools.partial(_mlp, nb=B, count=count),
        grid=(2 * B + 1,),
        in_specs=[
            pl.BlockSpec((Np, C0), lambda p: (jnp.minimum(p, B - 1), 0)),
            pl.BlockSpec((1, Np, D2), lambda p: (jnp.minimum(p, B - 1), 0, 0)),
            pl.BlockSpec((D1, C0), lambda p: (0, 0)),
            pl.BlockSpec((D2, C0), lambda p: (0, 0)),
            pl.BlockSpec((C0, C1), lambda p: (0, 0)),
            pl.BlockSpec((1, C0), lambda p: (0, 0)),
            pl.BlockSpec((1, C0), lambda p: (0, 0)),
            pl.BlockSpec((1, C1), lambda p: (0, 0)),
            pl.BlockSpec((1, C1), lambda p: (0, 0)),
        ],
        out_specs=pl.BlockSpec(
            (1, Np, C1), lambda p: (jnp.maximum(p - B - 1, 0), 0, 0)),
        out_shape=jax.ShapeDtypeStruct((B, Np, C1), jnp.float32),
        scratch_shapes=[
            pltpu.VMEM((B * Np, C0), jnp.float32),
            pltpu.VMEM((B * Np, C1), jnp.float32),
            pltpu.VMEM((1, C0), jnp.float32),
            pltpu.VMEM((1, C0), jnp.float32),
            pltpu.VMEM((1, C1), jnp.float32),
            pltpu.VMEM((1, C1), jnp.float32),
        ],
        compiler_params=pltpu.CompilerParams(
            dimension_semantics=("arbitrary",)),
    )(interp, features2, w0at, w0bt, w1t, g0r, beta0r, g1r, beta1r)

    return out


# speedup vs baseline: 5.2650x; 5.2650x over previous
import functools
import numpy as np
import jax
import jax.numpy as jnp
from jax import lax
from jax.experimental import pallas as pl
from jax.experimental.pallas import tpu as pltpu

_EPS = float(np.finfo(np.float32).eps)


def _fused(x2_ref, x1t_ref, f1_ref, f2_ref, w0at_ref, w0bt_ref, w1t_ref,
           g0_ref, be0_ref, g1_ref, be1_ref, out_ref,
           y0_scr, y1_scr, s0_scr, ss0_scr, sc1_scr, sh1_scr,
           *, nb, count):
    p = pl.program_id(0)
    npts = x2_ref.shape[1]

    @pl.when(p < nb)
    def _phase_a():
        b = p
        x2 = x2_ref[0]
        x1t = x1t_ref[0]
        n = x1t.shape[1]

        cross = jnp.dot(x2, x1t, preferred_element_type=jnp.float32)
        n2 = jnp.sum(x2 * x2, axis=1, keepdims=True)
        n1 = jnp.sum(x1t * x1t, axis=0, keepdims=True)
        acc = jnp.maximum(n2 - (cross + cross) + n1, 0.0)

        inf = jnp.float32(np.inf)
        m1 = jnp.min(acc, axis=1, keepdims=True)
        d2 = jnp.where(acc == m1, inf, acc)
        m2 = jnp.min(d2, axis=1, keepdims=True)
        d3 = jnp.where(d2 == m2, inf, d2)
        m3 = jnp.min(d3, axis=1, keepdims=True)
        r1 = 1.0 / (m1 + _EPS)
        r2 = 1.0 / (m2 + _EPS)
        r3 = 1.0 / (m3 + _EPS)
        inv_tot = 1.0 / (r1 + r2 + r3)
        wmat = jnp.where(
            acc == m1, r1,
            jnp.where(acc == m2, r2, jnp.where(acc == m3, r3, 0.0)))

        interp = jnp.dot(wmat, f1_ref[0],
                         preferred_element_type=jnp.float32) * inv_tot
        y0 = jnp.dot(interp, w0at_ref[...], preferred_element_type=jnp.float32)
        y0 = y0 + jnp.dot(f2_ref[0], w0bt_ref[...],
                          preferred_element_type=jnp.float32)
        y0_scr[pl.ds(b * npts, npts)] = y0

        s = jnp.sum(y0, axis=0, keepdims=True)
        ss = jnp.sum(y0 * y0, axis=0, keepdims=True)

        @pl.when(b == 0)
        def _():
            s0_scr[...] = s
            ss0_scr[...] = ss

        @pl.when(b != 0)
        def _():
            s0_scr[...] += s
            ss0_scr[...] += ss

    @pl.when(p == nb)
    def _phase_b():
        mean = s0_scr[...] * (1.0 / count)
        var = ss0_scr[...] * (1.0 / count) - mean * mean
        inv = lax.rsqrt(var + 1e-5)
        scale = g0_ref[...] * inv
        shift = be0_ref[...] - mean * scale
        x = jnp.maximum(y0_scr[...] * scale + shift, 0.0)
        y1 = jnp.dot(x, w1t_ref[...], preferred_element_type=jnp.float32)
        y1_scr[...] = y1

        s1 = jnp.sum(y1, axis=0, keepdims=True)
        ss1 = jnp.sum(y1 * y1, axis=0, keepdims=True)
        mean1 = s1 * (1.0 / count)
        var1 = ss1 * (1.0 / count) - mean1 * mean1
        inv1 = lax.rsqrt(var1 + 1e-5)
        sc1_scr[...] = g1_ref[...] * inv1
        sh1_scr[...] = be1_ref[...] - mean1 * (g1_ref[...] * inv1)

    @pl.when(p > nb)
    def _phase_c():
        b = p - nb - 1
        out_ref[0] = jnp.maximum(
            y1_scr[pl.ds(b * npts, npts)] * sc1_scr[...] + sh1_scr[...], 0.0)


def kernel(xyz1, xyz2, features1, features2, W0, b0, g0, beta0,
           W1, b1, g1, beta1):
    B, N, _ = xyz1.shape
    Np = xyz2.shape[1]
    D1 = features1.shape[2]
    D2 = features2.shape[2]
    C0 = W0.shape[0]
    C1 = W1.shape[0]
    count = float(B * Np)

    x1t = xyz1.transpose(0, 2, 1)
    w0at = W0[:, :D1].T
    w0bt = W0[:, D1:].T
    w1t = W1.T
    g0r = g0.reshape(1, C0)
    beta0r = beta0.reshape(1, C0)
    g1r = g1.reshape(1, C1)
    beta1r = beta1.reshape(1, C1)

    def a_block(shape):
        return pl.BlockSpec(
            (1,) + shape,
            lambda p: (jnp.minimum(p, B - 1),) + (0,) * len(shape))

    def fixed_block(shape):
        return pl.BlockSpec(shape, lambda p: (0,) * len(shape))

    out = pl.pallas_call(
        functools.partial(_fused, nb=B, count=count),
        grid=(2 * B + 1,),
        in_specs=[
            a_block((Np, 3)),
            a_block((3, N)),
            a_block((N, D1)),
            a_block((Np, D2)),
            fixed_block((D1, C0)),
            fixed_block((D2, C0)),
            fixed_block((C0, C1)),
            fixed_block((1, C0)),
            fixed_block((1, C0)),
            fixed_block((1, C1)),
            fixed_block((1, C1)),
        ],
        out_specs=pl.BlockSpec(
            (1, Np, C1),
            lambda p: (jnp.maximum(p - B - 1, 0), 0, 0)),
        out_shape=jax.ShapeDtypeStruct((B, Np, C1), jnp.float32),
        scratch_shapes=[
            pltpu.VMEM((B * Np, C0), jnp.float32),
            pltpu.VMEM((B * Np, C1), jnp.float32),
            pltpu.VMEM((1, C0), jnp.float32),
            pltpu.VMEM((1, C0), jnp.float32),
            pltpu.VMEM((1, C1), jnp.float32),
            pltpu.VMEM((1, C1), jnp.float32),
        ],
        compiler_params=pltpu.CompilerParams(
            dimension_semantics=("arbitrary",)),
    )(xyz2, x1t, features1, features2, w0at, w0bt, w1t,
      g0r, beta0r, g1r, beta1r)

    return out
